# baseline (speedup 1.0000x reference)
import functools

import jax
import jax.numpy as jnp
from jax import lax
from jax.experimental import pallas as pl
from jax.experimental.pallas import tpu as pltpu
from jax.experimental.pallas import tpu_sc as plsc

N = 10000
E = 160000
NG = 128
OUT = 10

NC = 2
NS = 16
EB = 128
TB = E // EB
NBT = TB // (NC * NS)
NBR = TB % (NC * NS)
SB = 48
TBP = 1280
NBUF = 2
KMAX = (NBT + 1 + NBUF - 1) // NBUF
RPT = 624
RTAIL = N - RPT * NS

BLK = 1000


@functools.lru_cache(maxsize=None)
def _seg_sum(nch):
    mesh = plsc.VectorSubcoreMesh(core_axis_name="c", subcore_axis_name="s")

    @functools.partial(
        pl.kernel,
        out_type=jax.ShapeDtypeStruct((nch, NC, N, 128), jnp.float32),
        mesh=mesh,
        scratch_types=[
            pltpu.VMEM((SB, EB), jnp.int32),
            pltpu.VMEM((SB, EB), jnp.int32),
            [pltpu.VMEM((EB, 128), jnp.float32) for _ in range(NBUF)],
            [pltpu.SemaphoreType.DMA for _ in range(NBUF)],
            [pltpu.SemaphoreType.DMA for _ in range(NBUF)],
            pltpu.VMEM_SHARED((N, 128), jnp.float32),
        ],
    )
    def seg_sum(h_hbm, src_hbm, dst_hbm, zeros_hbm, out_hbm,
                src_v, dst_v, rows, gsem, ssem, agg_sh):
        cid = lax.axis_index("c")
        sid = lax.axis_index("s")
        wid = cid * NS + sid
        fb = NBT * wid + jnp.minimum(wid, NBR)
        nb = NBT + (wid < NBR).astype(jnp.int32)
        fb8 = (fb // 8) * 8
        sh = fb - fb8
        rbase = sid * RPT

        pltpu.sync_copy(src_hbm.at[pl.ds(fb8, SB)], src_v)
        pltpu.sync_copy(dst_hbm.at[pl.ds(fb8, SB)], dst_v)

        for ch in range(nch):
            h_ch = h_hbm.at[ch]
            pltpu.sync_copy(zeros_hbm.at[pl.ds(rbase, RPT)],
                            agg_sh.at[pl.ds(rbase, RPT)])

            @pl.when(sid == 0)
            def _():
                pltpu.sync_copy(zeros_hbm.at[pl.ds(RPT * NS, RTAIL)],
                                agg_sh.at[pl.ds(RPT * NS, RTAIL)])

            plsc.subcore_barrier()

            for p in range(NBUF):
                pltpu.async_copy(h_ch.at[src_v.at[sh + p]], rows[p],
                                 gsem[p])

            def body(k, carry):
                for p in range(NBUF):
                    b = NBUF * k + p

                    @pl.when(b < nb)
                    def _():
                        pltpu.make_async_copy(h_ch.at[src_v.at[sh]],
                                              rows[p], gsem[p]).wait()
                        pltpu.async_copy(rows[p],
                                         agg_sh.at[dst_v.at[sh + b]],
                                         ssem[p], add=True)

                        @pl.when(b + NBUF < nb)
                        def _():
                            pltpu.make_async_copy(
                                rows[p], agg_sh.at[dst_v.at[sh]],
                                ssem[p]).wait()
                            pltpu.async_copy(
                                h_ch.at[src_v.at[sh + b + NBUF]], rows[p],
                                gsem[p])

                return carry

            lax.fori_loop(0, KMAX, body, 0)
            for p in range(NBUF):
                pltpu.make_async_copy(rows[p], agg_sh.at[dst_v.at[sh]],
                                      ssem[p]).wait()
            plsc.subcore_barrier()
            pltpu.sync_copy(agg_sh.at[pl.ds(rbase, RPT)],
                            out_hbm.at[ch].at[cid].at[pl.ds(rbase, RPT)])

            @pl.when(sid == 0)
            def _():
                pltpu.sync_copy(agg_sh.at[pl.ds(RPT * NS, RTAIL)],
                                out_hbm.at[ch].at[cid].at[pl.ds(RPT * NS,
                                                                RTAIL)])

    return seg_sum


@functools.lru_cache(maxsize=None)
def _mlp(nch_in, dh, nch_out):
    din = nch_in * 128
    ni = N // BLK

    def body(h_ref, a_ref, w1_ref, b1_ref, g_ref, be_ref, w2_ref, b2_ref,
             w3_ref, b3_ref, o_ref, z_s, s_s, q_s):
        p = pl.program_id(0)
        i = pl.program_id(1)

        @pl.when(p == 0)
        def _():
            g = jnp.concatenate(
                [h_ref[c] + a_ref[2 * c] + a_ref[2 * c + 1]
                 for c in range(nch_in)], axis=1)
            z = jnp.dot(g, w1_ref[...], preferred_element_type=jnp.float32)
            z = z + b1_ref[...]
            z_s[pl.ds(i * BLK, BLK), :] = z
            s = jnp.sum(z, axis=0, keepdims=True)
            q = jnp.sum(z * z, axis=0, keepdims=True)

            @pl.when(i == 0)
            def _():
                s_s[...] = s
                q_s[...] = q

            @pl.when(i != 0)
            def _():
                s_s[...] += s
                q_s[...] += q

        @pl.when(p == 1)
        def _():
            mean = s_s[...] * (1.0 / N)
            var = q_s[...] * (1.0 / N) - mean * mean
            inv = g_ref[...] * lax.rsqrt(var + 1e-5)
            z = z_s[pl.ds(i * BLK, BLK), :]
            z = jnp.maximum(z * inv + (be_ref[...] - mean * inv), 0.0)
            h2 = jnp.dot(z, w2_ref[...], preferred_element_type=jnp.float32)
            h2 = jnp.maximum(h2 + b2_ref[...], 0.0)
            h3 = jnp.dot(h2, w3_ref[...], preferred_element_type=jnp.float32)
            h3 = jnp.maximum(h3 + b3_ref[...], 0.0)
            for c in range(nch_out):
                o_ref[c] = h3[:, c * 128:(c + 1) * 128]

    def first_only(p, i):
        return (0, jnp.where(p == 0, i, 0), 0)

    def const(p, i):
        return (0, 0)

    return pl.pallas_call(
        body,
        grid=(2, ni),
        in_specs=[
            pl.BlockSpec((nch_in, BLK, 128), first_only),
            pl.BlockSpec((nch_in * NC, BLK, 128), first_only),
            pl.BlockSpec((din, dh), const),
            pl.BlockSpec((1, dh), const),
            pl.BlockSpec((1, dh), const),
            pl.BlockSpec((1, dh), const),
            pl.BlockSpec((dh, dh), const),
            pl.BlockSpec((1, dh), const),
            pl.BlockSpec((dh, dh), const),
            pl.BlockSpec((1, dh), const),
        ],
        out_specs=pl.BlockSpec((nch_out, BLK, 128),
                               lambda p, i: (0, jnp.where(p == 0, 0, i), 0)),
        out_shape=jax.ShapeDtypeStruct((nch_out, N, 128), jnp.float32),
        scratch_shapes=[
            pltpu.VMEM((N, dh), jnp.float32),
            pltpu.VMEM((1, dh), jnp.float32),
            pltpu.VMEM((1, dh), jnp.float32),
        ],
    )


def _pool_head():
    def body(h_ref, b_ref, w1_ref, b1_ref, w2_ref, b2_ref, o_ref, acc):
        i = pl.program_id(0)
        bids = b_ref[0]
        oh = (lax.broadcasted_iota(jnp.int32, (NG, BLK), 0) == bids)
        oh = oh.astype(jnp.float32)
        p = jnp.dot(oh, h_ref[0], preferred_element_type=jnp.float32)

        @pl.when(i == 0)
        def _():
            acc[...] = p

        @pl.when(i != 0)
        def _():
            acc[...] += p

        @pl.when(i == N // BLK - 1)
        def _():
            t = jnp.dot(acc[...], w1_ref[...],
                        preferred_element_type=jnp.float32)
            t = jnp.maximum(t + b1_ref[...], 0.0)
            o_ref[...] = jnp.dot(t, w2_ref[...],
                                 preferred_element_type=jnp.float32) + b2_ref[...]

    return pl.pallas_call(
        body,
        grid=(N // BLK,),
        in_specs=[
            pl.BlockSpec((1, BLK, 128), lambda i: (0, i, 0)),
            pl.BlockSpec((1, 1, BLK), lambda i: (i, 0, 0)),
            pl.BlockSpec((NG, 64), lambda i: (0, 0)),
            pl.BlockSpec((1, 64), lambda i: (0, 0)),
            pl.BlockSpec((64, 128), lambda i: (0, 0)),
            pl.BlockSpec((1, 128), lambda i: (0, 0)),
        ],
        out_specs=pl.BlockSpec((NG, 128), lambda i: (0, 0)),
        out_shape=jax.ShapeDtypeStruct((NG, 128), jnp.float32),
        scratch_shapes=[pltpu.VMEM((NG, NG), jnp.float32)],
    )


_LAYER_DIMS = [(1, 128), (1, 256), (2, 512), (4, 256), (2, 128)]


def kernel(x, edge_index, batch, params):
    src = jnp.pad(edge_index[0].reshape(TB, EB), ((0, TBP - TB), (0, 0)))
    dst = jnp.pad(edge_index[1].reshape(TB, EB), ((0, TBP - TB), (0, 0)))
    zeros = jnp.zeros((N, 128), jnp.float32)
    xp = jnp.pad(x, ((0, 0), (0, 2)))
    h = xp.reshape(1, N, 128)

    for li, (nch_in, dh) in enumerate(_LAYER_DIMS):
        p = params[f"conv{li + 1}"]
        w1 = p["W1"]
        if li == 0:
            w1 = jnp.pad(w1, ((0, 2), (0, 0)))
        agg = _seg_sum(nch_in)(h, src, dst, zeros)
        h = _mlp(nch_in, dh, dh // 128)(
            h, agg.reshape(nch_in * NC, N, 128), w1, p["b1"].reshape(1, dh),
            p["gamma"].reshape(1, dh), p["beta"].reshape(1, dh),
            p["W2"], p["b2"].reshape(1, dh),
            p["W3"], p["b3"].reshape(1, dh))

    w2p = jnp.pad(params["lin2_W"], ((0, 0), (0, 128 - OUT)))
    b2p = jnp.pad(params["lin2_b"], ((0, 128 - OUT),)).reshape(1, 128)
    out = _pool_head()(
        h, batch.reshape(N // BLK, 1, BLK),
        params["lin1_W"], params["lin1_b"].reshape(1, 64),
        w2p, b2p)
    return out[:, :OUT]


# speedup vs baseline: 1.0577x; 1.0577x over previous
import functools

import jax
import jax.numpy as jnp
from jax import lax
from jax.experimental import pallas as pl
from jax.experimental.pallas import tpu as pltpu
from jax.experimental.pallas import tpu_sc as plsc

N = 10000
E = 160000
NG = 128
OUT = 10

NC = 2
NS = 16
EB = 80
TB = E // EB
NBT = TB // (NC * NS)
NBR = TB % (NC * NS)
SB = 72
TBP = 2016
NBUF = 3
KMAX = (NBT + 1 + NBUF - 1) // NBUF
RPT = 624
RTAIL = N - RPT * NS

BLK = 1000


@functools.lru_cache(maxsize=None)
def _seg_sum(nch):
    mesh = plsc.VectorSubcoreMesh(core_axis_name="c", subcore_axis_name="s")

    @functools.partial(
        pl.kernel,
        out_type=jax.ShapeDtypeStruct((nch, NC, N, 128), jnp.float32),
        mesh=mesh,
        scratch_types=[
            pltpu.VMEM((SB, EB), jnp.int32),
            pltpu.VMEM((SB, EB), jnp.int32),
            [pltpu.VMEM((EB, 128), jnp.float32) for _ in range(NBUF)],
            [pltpu.SemaphoreType.DMA for _ in range(NBUF)],
            [pltpu.SemaphoreType.DMA for _ in range(NBUF)],
            pltpu.VMEM_SHARED((N, 128), jnp.float32),
        ],
    )
    def seg_sum(h_hbm, src_hbm, dst_hbm, zeros_hbm, out_hbm,
                src_v, dst_v, rows, gsem, ssem, agg_sh):
        cid = lax.axis_index("c")
        sid = lax.axis_index("s")
        wid = cid * NS + sid
        fb = NBT * wid + jnp.minimum(wid, NBR)
        nb = NBT + (wid < NBR).astype(jnp.int32)
        fb8 = (fb // 8) * 8
        sh = fb - fb8
        rbase = sid * RPT

        pltpu.sync_copy(src_hbm.at[pl.ds(fb8, SB)], src_v)
        pltpu.sync_copy(dst_hbm.at[pl.ds(fb8, SB)], dst_v)

        for ch in range(nch):
            h_ch = h_hbm.at[ch]
            pltpu.sync_copy(zeros_hbm.at[pl.ds(rbase, RPT)],
                            agg_sh.at[pl.ds(rbase, RPT)])

            @pl.when(sid == 0)
            def _():
                pltpu.sync_copy(zeros_hbm.at[pl.ds(RPT * NS, RTAIL)],
                                agg_sh.at[pl.ds(RPT * NS, RTAIL)])

            plsc.subcore_barrier()

            for p in range(NBUF):
                pltpu.async_copy(h_ch.at[src_v.at[sh + p]], rows[p],
                                 gsem[p])

            def body(k, carry):
                for p in range(NBUF):
                    b = NBUF * k + p

                    @pl.when(b < nb)
                    def _():
                        pltpu.make_async_copy(h_ch.at[src_v.at[sh]],
                                              rows[p], gsem[p]).wait()
                        pltpu.async_copy(rows[p],
                                         agg_sh.at[dst_v.at[sh + b]],
                                         ssem[p], add=True)

                        @pl.when(b + NBUF < nb)
                        def _():
                            pltpu.make_async_copy(
                                rows[p], agg_sh.at[dst_v.at[sh]],
                                ssem[p]).wait()
                            pltpu.async_copy(
                                h_ch.at[src_v.at[sh + b + NBUF]], rows[p],
                                gsem[p])

                return carry

            lax.fori_loop(0, KMAX, body, 0)
            for p in range(NBUF):
                pltpu.make_async_copy(rows[p], agg_sh.at[dst_v.at[sh]],
                                      ssem[p]).wait()
            plsc.subcore_barrier()
            pltpu.sync_copy(agg_sh.at[pl.ds(rbase, RPT)],
                            out_hbm.at[ch].at[cid].at[pl.ds(rbase, RPT)])

            @pl.when(sid == 0)
            def _():
                pltpu.sync_copy(agg_sh.at[pl.ds(RPT * NS, RTAIL)],
                                out_hbm.at[ch].at[cid].at[pl.ds(RPT * NS,
                                                                RTAIL)])

    return seg_sum


@functools.lru_cache(maxsize=None)
def _mlp(nch_in, dh, nch_out):
    din = nch_in * 128
    ni = N // BLK

    def body(h_ref, a_ref, w1_ref, b1_ref, g_ref, be_ref, w2_ref, b2_ref,
             w3_ref, b3_ref, o_ref, z_s, s_s, q_s):
        p = pl.program_id(0)
        i = pl.program_id(1)

        @pl.when(p == 0)
        def _():
            g = jnp.concatenate(
                [h_ref[c] + a_ref[2 * c] + a_ref[2 * c + 1]
                 for c in range(nch_in)], axis=1)
            z = jnp.dot(g, w1_ref[...], preferred_element_type=jnp.float32)
            z = z + b1_ref[...]
            z_s[pl.ds(i * BLK, BLK), :] = z
            s = jnp.sum(z, axis=0, keepdims=True)
            q = jnp.sum(z * z, axis=0, keepdims=True)

            @pl.when(i == 0)
            def _():
                s_s[...] = s
                q_s[...] = q

            @pl.when(i != 0)
            def _():
                s_s[...] += s
                q_s[...] += q

        @pl.when(p == 1)
        def _():
            mean = s_s[...] * (1.0 / N)
            var = q_s[...] * (1.0 / N) - mean * mean
            inv = g_ref[...] * lax.rsqrt(var + 1e-5)
            z = z_s[pl.ds(i * BLK, BLK), :]
            z = jnp.maximum(z * inv + (be_ref[...] - mean * inv), 0.0)
            h2 = jnp.dot(z, w2_ref[...], preferred_element_type=jnp.float32)
            h2 = jnp.maximum(h2 + b2_ref[...], 0.0)
            h3 = jnp.dot(h2, w3_ref[...], preferred_element_type=jnp.float32)
            h3 = jnp.maximum(h3 + b3_ref[...], 0.0)
            for c in range(nch_out):
                o_ref[c] = h3[:, c * 128:(c + 1) * 128]

    def first_only(p, i):
        return (0, jnp.where(p == 0, i, 0), 0)

    def const(p, i):
        return (0, 0)

    return pl.pallas_call(
        body,
        grid=(2, ni),
        in_specs=[
            pl.BlockSpec((nch_in, BLK, 128), first_only),
            pl.BlockSpec((nch_in * NC, BLK, 128), first_only),
            pl.BlockSpec((din, dh), const),
            pl.BlockSpec((1, dh), const),
            pl.BlockSpec((1, dh), const),
            pl.BlockSpec((1, dh), const),
            pl.BlockSpec((dh, dh), const),
            pl.BlockSpec((1, dh), const),
            pl.BlockSpec((dh, dh), const),
            pl.BlockSpec((1, dh), const),
        ],
        out_specs=pl.BlockSpec((nch_out, BLK, 128),
                               lambda p, i: (0, jnp.where(p == 0, 0, i), 0)),
        out_shape=jax.ShapeDtypeStruct((nch_out, N, 128), jnp.float32),
        scratch_shapes=[
            pltpu.VMEM((N, dh), jnp.float32),
            pltpu.VMEM((1, dh), jnp.float32),
            pltpu.VMEM((1, dh), jnp.float32),
        ],
    )


def _pool_head():
    def body(h_ref, b_ref, w1_ref, b1_ref, w2_ref, b2_ref, o_ref, acc):
        i = pl.program_id(0)
        bids = b_ref[0]
        oh = (lax.broadcasted_iota(jnp.int32, (NG, BLK), 0) == bids)
        oh = oh.astype(jnp.float32)
        p = jnp.dot(oh, h_ref[0], preferred_element_type=jnp.float32)

        @pl.when(i == 0)
        def _():
            acc[...] = p

        @pl.when(i != 0)
        def _():
            acc[...] += p

        @pl.when(i == N // BLK - 1)
        def _():
            t = jnp.dot(acc[...], w1_ref[...],
                        preferred_element_type=jnp.float32)
            t = jnp.maximum(t + b1_ref[...], 0.0)
            o_ref[...] = jnp.dot(t, w2_ref[...],
                                 preferred_element_type=jnp.float32) + b2_ref[...]

    return pl.pallas_call(
        body,
        grid=(N // BLK,),
        in_specs=[
            pl.BlockSpec((1, BLK, 128), lambda i: (0, i, 0)),
            pl.BlockSpec((1, 1, BLK), lambda i: (i, 0, 0)),
            pl.BlockSpec((NG, 64), lambda i: (0, 0)),
            pl.BlockSpec((1, 64), lambda i: (0, 0)),
            pl.BlockSpec((64, 128), lambda i: (0, 0)),
            pl.BlockSpec((1, 128), lambda i: (0, 0)),
        ],
        out_specs=pl.BlockSpec((NG, 128), lambda i: (0, 0)),
        out_shape=jax.ShapeDtypeStruct((NG, 128), jnp.float32),
        scratch_shapes=[pltpu.VMEM((NG, NG), jnp.float32)],
    )


_LAYER_DIMS = [(1, 128), (1, 256), (2, 512), (4, 256), (2, 128)]


def kernel(x, edge_index, batch, params):
    src = jnp.pad(edge_index[0].reshape(TB, EB), ((0, TBP - TB), (0, 0)))
    dst = jnp.pad(edge_index[1].reshape(TB, EB), ((0, TBP - TB), (0, 0)))
    zeros = jnp.zeros((N, 128), jnp.float32)
    xp = jnp.pad(x, ((0, 0), (0, 2)))
    h = xp.reshape(1, N, 128)

    for li, (nch_in, dh) in enumerate(_LAYER_DIMS):
        p = params[f"conv{li + 1}"]
        w1 = p["W1"]
        if li == 0:
            w1 = jnp.pad(w1, ((0, 2), (0, 0)))
        agg = _seg_sum(nch_in)(h, src, dst, zeros)
        h = _mlp(nch_in, dh, dh // 128)(
            h, agg.reshape(nch_in * NC, N, 128), w1, p["b1"].reshape(1, dh),
            p["gamma"].reshape(1, dh), p["beta"].reshape(1, dh),
            p["W2"], p["b2"].reshape(1, dh),
            p["W3"], p["b3"].reshape(1, dh))

    w2p = jnp.pad(params["lin2_W"], ((0, 0), (0, 128 - OUT)))
    b2p = jnp.pad(params["lin2_b"], ((0, 128 - OUT),)).reshape(1, 128)
    out = _pool_head()(
        h, batch.reshape(N // BLK, 1, BLK),
        params["lin1_W"], params["lin1_b"].reshape(1, 64),
        w2p, b2p)
    return out[:, :OUT]


# speedup vs baseline: 1.0591x; 1.0014x over previous
import functools

import jax
import jax.numpy as jnp
from jax import lax
from jax.experimental import pallas as pl
from jax.experimental.pallas import tpu as pltpu
from jax.experimental.pallas import tpu_sc as plsc

N = 10000
E = 160000
NG = 128
OUT = 10

NC = 2
NS = 16
EB = 80
TB = E // EB
NBT = TB // (NC * NS)
NBR = TB % (NC * NS)
SB = 72
TBP = 2016
NBUF = 3
KMAX = (NBT + 1 + NBUF - 1) // NBUF
RPT = 624
RTAIL = N - RPT * NS

BLK = 1000


@functools.lru_cache(maxsize=None)
def _seg_sum(nch):
    mesh = plsc.VectorSubcoreMesh(core_axis_name="c", subcore_axis_name="s")

    @functools.partial(
        pl.kernel,
        out_type=jax.ShapeDtypeStruct((nch, NC, N, 128), jnp.float32),
        mesh=mesh,
        scratch_types=[
            pltpu.VMEM((SB, EB), jnp.int32),
            pltpu.VMEM((SB, EB), jnp.int32),
            [pltpu.VMEM((EB, 128), jnp.float32) for _ in range(NBUF)],
            [pltpu.SemaphoreType.DMA for _ in range(NBUF)],
            [pltpu.SemaphoreType.DMA for _ in range(NBUF)],
            pltpu.VMEM_SHARED((N, 128), jnp.float32),
        ],
    )
    def seg_sum(h_hbm, src_hbm, dst_hbm, zeros_hbm, out_hbm,
                src_v, dst_v, rows, gsem, ssem, agg_sh):
        cid = lax.axis_index("c")
        sid = lax.axis_index("s")
        wid = cid * NS + sid
        fb = NBT * wid + jnp.minimum(wid, NBR)
        nb = NBT + (wid < NBR).astype(jnp.int32)
        fb8 = (fb // 8) * 8
        sh = fb - fb8
        rbase = sid * RPT

        pltpu.sync_copy(src_hbm.at[pl.ds(fb8, SB)], src_v)
        pltpu.sync_copy(dst_hbm.at[pl.ds(fb8, SB)], dst_v)

        for ch in range(nch):
            h_ch = h_hbm.at[ch]
            pltpu.sync_copy(zeros_hbm.at[pl.ds(rbase, RPT)],
                            agg_sh.at[pl.ds(rbase, RPT)])

            @pl.when(sid == 0)
            def _():
                pltpu.sync_copy(zeros_hbm.at[pl.ds(RPT * NS, RTAIL)],
                                agg_sh.at[pl.ds(RPT * NS, RTAIL)])

            plsc.subcore_barrier()

            for p in range(NBUF):
                pltpu.async_copy(h_ch.at[src_v.at[sh + p]], rows[p],
                                 gsem[p])

            def body(k, carry):
                for p in range(NBUF):
                    b = NBUF * k + p

                    @pl.when(b < nb)
                    def _():
                        pltpu.make_async_copy(h_ch.at[src_v.at[sh]],
                                              rows[p], gsem[p]).wait()
                        pltpu.async_copy(rows[p],
                                         agg_sh.at[dst_v.at[sh + b]],
                                         ssem[p], add=True)

                        @pl.when(b + NBUF < nb)
                        def _():
                            pltpu.make_async_copy(
                                rows[p], agg_sh.at[dst_v.at[sh]],
                                ssem[p]).wait()
                            pltpu.async_copy(
                                h_ch.at[src_v.at[sh + b + NBUF]], rows[p],
                                gsem[p])

                return carry

            lax.fori_loop(0, KMAX, body, 0)
            for p in range(NBUF):
                pltpu.make_async_copy(rows[p], agg_sh.at[dst_v.at[sh]],
                                      ssem[p]).wait()
            plsc.subcore_barrier()
            pltpu.sync_copy(agg_sh.at[pl.ds(rbase, RPT)],
                            out_hbm.at[ch].at[cid].at[pl.ds(rbase, RPT)])

            @pl.when(sid == 0)
            def _():
                pltpu.sync_copy(agg_sh.at[pl.ds(RPT * NS, RTAIL)],
                                out_hbm.at[ch].at[cid].at[pl.ds(RPT * NS,
                                                                RTAIL)])

    return seg_sum


@functools.lru_cache(maxsize=None)
def _mlp(nch_in, dh, nch_out):
    din = nch_in * 128
    ni = N // BLK

    def body(h_ref, a_ref, w1_ref, b1_ref, g_ref, be_ref, w2_ref, b2_ref,
             w3_ref, b3_ref, o_ref, z_s, s_s, q_s):
        p = pl.program_id(0)
        i = pl.program_id(1)

        @pl.when(p == 0)
        def _():
            g = jnp.concatenate(
                [h_ref[c] + a_ref[2 * c] + a_ref[2 * c + 1]
                 for c in range(nch_in)], axis=1)
            z = jnp.dot(g.astype(jnp.bfloat16), w1_ref[...],
                        preferred_element_type=jnp.float32)
            z = z + b1_ref[...]
            z_s[pl.ds(i * BLK, BLK), :] = z
            s = jnp.sum(z, axis=0, keepdims=True)
            q = jnp.sum(z * z, axis=0, keepdims=True)

            @pl.when(i == 0)
            def _():
                s_s[...] = s
                q_s[...] = q

            @pl.when(i != 0)
            def _():
                s_s[...] += s
                q_s[...] += q

        @pl.when(p == 1)
        def _():
            mean = s_s[...] * (1.0 / N)
            var = q_s[...] * (1.0 / N) - mean * mean
            inv = g_ref[...] * lax.rsqrt(var + 1e-5)
            z = z_s[pl.ds(i * BLK, BLK), :]
            z = jnp.maximum(z * inv + (be_ref[...] - mean * inv), 0.0)
            h2 = jnp.dot(z.astype(jnp.bfloat16), w2_ref[...],
                         preferred_element_type=jnp.float32)
            h2 = jnp.maximum(h2 + b2_ref[...], 0.0)
            h3 = jnp.dot(h2.astype(jnp.bfloat16), w3_ref[...],
                         preferred_element_type=jnp.float32)
            h3 = jnp.maximum(h3 + b3_ref[...], 0.0)
            for c in range(nch_out):
                o_ref[c] = h3[:, c * 128:(c + 1) * 128]

    def first_only(p, i):
        return (0, jnp.where(p == 0, i, 0), 0)

    def const(p, i):
        return (0, 0)

    return pl.pallas_call(
        body,
        grid=(2, ni),
        in_specs=[
            pl.BlockSpec((nch_in, BLK, 128), first_only),
            pl.BlockSpec((nch_in * NC, BLK, 128), first_only),
            pl.BlockSpec((din, dh), const),
            pl.BlockSpec((1, dh), const),
            pl.BlockSpec((1, dh), const),
            pl.BlockSpec((1, dh), const),
            pl.BlockSpec((dh, dh), const),
            pl.BlockSpec((1, dh), const),
            pl.BlockSpec((dh, dh), const),
            pl.BlockSpec((1, dh), const),
        ],
        out_specs=pl.BlockSpec((nch_out, BLK, 128),
                               lambda p, i: (0, jnp.where(p == 0, 0, i), 0)),
        out_shape=jax.ShapeDtypeStruct((nch_out, N, 128), jnp.float32),
        scratch_shapes=[
            pltpu.VMEM((N, dh), jnp.float32),
            pltpu.VMEM((1, dh), jnp.float32),
            pltpu.VMEM((1, dh), jnp.float32),
        ],
    )


def _pool_head():
    def body(h_ref, b_ref, w1_ref, b1_ref, w2_ref, b2_ref, o_ref, acc):
        i = pl.program_id(0)
        bids = b_ref[0]
        oh = (lax.broadcasted_iota(jnp.int32, (NG, BLK), 0) == bids)
        oh = oh.astype(jnp.float32)
        p = jnp.dot(oh, h_ref[0], preferred_element_type=jnp.float32)

        @pl.when(i == 0)
        def _():
            acc[...] = p

        @pl.when(i != 0)
        def _():
            acc[...] += p

        @pl.when(i == N // BLK - 1)
        def _():
            t = jnp.dot(acc[...], w1_ref[...],
                        preferred_element_type=jnp.float32)
            t = jnp.maximum(t + b1_ref[...], 0.0)
            o_ref[...] = jnp.dot(t, w2_ref[...],
                                 preferred_element_type=jnp.float32) + b2_ref[...]

    return pl.pallas_call(
        body,
        grid=(N // BLK,),
        in_specs=[
            pl.BlockSpec((1, BLK, 128), lambda i: (0, i, 0)),
            pl.BlockSpec((1, 1, BLK), lambda i: (i, 0, 0)),
            pl.BlockSpec((NG, 64), lambda i: (0, 0)),
            pl.BlockSpec((1, 64), lambda i: (0, 0)),
            pl.BlockSpec((64, 128), lambda i: (0, 0)),
            pl.BlockSpec((1, 128), lambda i: (0, 0)),
        ],
        out_specs=pl.BlockSpec((NG, 128), lambda i: (0, 0)),
        out_shape=jax.ShapeDtypeStruct((NG, 128), jnp.float32),
        scratch_shapes=[pltpu.VMEM((NG, NG), jnp.float32)],
    )


_LAYER_DIMS = [(1, 128), (1, 256), (2, 512), (4, 256), (2, 128)]


def kernel(x, edge_index, batch, params):
    src = jnp.pad(edge_index[0].reshape(TB, EB), ((0, TBP - TB), (0, 0)))
    dst = jnp.pad(edge_index[1].reshape(TB, EB), ((0, TBP - TB), (0, 0)))
    zeros = jnp.zeros((N, 128), jnp.float32)
    xp = jnp.pad(x, ((0, 0), (0, 2)))
    h = xp.reshape(1, N, 128)

    for li, (nch_in, dh) in enumerate(_LAYER_DIMS):
        p = params[f"conv{li + 1}"]
        w1 = p["W1"]
        if li == 0:
            w1 = jnp.pad(w1, ((0, 2), (0, 0)))
        agg = _seg_sum(nch_in)(h, src, dst, zeros)
        h = _mlp(nch_in, dh, dh // 128)(
            h, agg.reshape(nch_in * NC, N, 128),
            w1.astype(jnp.bfloat16), p["b1"].reshape(1, dh),
            p["gamma"].reshape(1, dh), p["beta"].reshape(1, dh),
            p["W2"].astype(jnp.bfloat16), p["b2"].reshape(1, dh),
            p["W3"].astype(jnp.bfloat16), p["b3"].reshape(1, dh))

    w2p = jnp.pad(params["lin2_W"], ((0, 0), (0, 128 - OUT)))
    b2p = jnp.pad(params["lin2_b"], ((0, 128 - OUT),)).reshape(1, 128)
    out = _pool_head()(
        h, batch.reshape(N // BLK, 1, BLK),
        params["lin1_W"], params["lin1_b"].reshape(1, 64),
        w2p, b2p)
    return out[:, :OUT]


# speedup vs baseline: 1.0869x; 1.0262x over previous
import functools

import jax
import jax.numpy as jnp
from jax import lax
from jax.experimental import pallas as pl
from jax.experimental.pallas import tpu as pltpu
from jax.experimental.pallas import tpu_sc as plsc

N = 10000
E = 160000
NG = 128
OUT = 10

NC = 2
NS = 16
EB = 80
TB = E // EB
NBT = TB // (NC * NS)
NBR = TB % (NC * NS)
SB = 72
TBP = 2016
NBUF = 3
KMAX = (NBT + 1 + NBUF - 1) // NBUF
RPT = 624
RTAIL = N - RPT * NS

BLK = 1000


@functools.lru_cache(maxsize=None)
def _seg_sum(nch):
    mesh = plsc.VectorSubcoreMesh(core_axis_name="c", subcore_axis_name="s")

    @functools.partial(
        pl.kernel,
        out_type=jax.ShapeDtypeStruct((nch, NC, N, 128), jnp.float32),
        mesh=mesh,
        scratch_types=[
            pltpu.VMEM((SB, EB), jnp.int32),
            pltpu.VMEM((SB, EB), jnp.int32),
            [pltpu.VMEM((EB, 128), jnp.float32) for _ in range(NBUF)],
            [pltpu.SemaphoreType.DMA for _ in range(NBUF)],
            [pltpu.SemaphoreType.DMA for _ in range(NBUF)],
            pltpu.VMEM_SHARED((N, 128), jnp.float32),
        ],
    )
    def seg_sum(h_hbm, src_hbm, dst_hbm, zeros_hbm, out_hbm,
                src_v, dst_v, rows, gsem, ssem, agg_sh):
        cid = lax.axis_index("c")
        sid = lax.axis_index("s")
        wid = cid * NS + sid
        fb = NBT * wid + jnp.minimum(wid, NBR)
        nb = NBT + (wid < NBR).astype(jnp.int32)
        fb8 = (fb // 8) * 8
        sh = fb - fb8
        rbase = sid * RPT

        pltpu.sync_copy(src_hbm.at[pl.ds(fb8, SB)], src_v)
        pltpu.sync_copy(dst_hbm.at[pl.ds(fb8, SB)], dst_v)

        def copyout(ch):
            pltpu.sync_copy(agg_sh.at[pl.ds(rbase, RPT)],
                            out_hbm.at[ch].at[cid].at[pl.ds(rbase, RPT)])

            @pl.when(sid == 0)
            def _():
                pltpu.sync_copy(agg_sh.at[pl.ds(RPT * NS, RTAIL)],
                                out_hbm.at[ch].at[cid].at[pl.ds(RPT * NS,
                                                                RTAIL)])

        for ch in range(nch):
            h_ch = h_hbm.at[ch]
            for p in range(NBUF):
                pltpu.async_copy(h_ch.at[src_v.at[sh + p]], rows[p],
                                 gsem[p])
            if ch > 0:
                copyout(ch - 1)
            pltpu.sync_copy(zeros_hbm.at[pl.ds(rbase, RPT)],
                            agg_sh.at[pl.ds(rbase, RPT)])

            @pl.when(sid == 0)
            def _():
                pltpu.sync_copy(zeros_hbm.at[pl.ds(RPT * NS, RTAIL)],
                                agg_sh.at[pl.ds(RPT * NS, RTAIL)])

            plsc.subcore_barrier()

            def body(k, carry):
                for p in range(NBUF):
                    b = NBUF * k + p

                    @pl.when(b < nb)
                    def _():
                        pltpu.make_async_copy(h_ch.at[src_v.at[sh]],
                                              rows[p], gsem[p]).wait()
                        pltpu.async_copy(rows[p],
                                         agg_sh.at[dst_v.at[sh + b]],
                                         ssem[p], add=True)

                        @pl.when(b + NBUF < nb)
                        def _():
                            pltpu.make_async_copy(
                                rows[p], agg_sh.at[dst_v.at[sh]],
                                ssem[p]).wait()
                            pltpu.async_copy(
                                h_ch.at[src_v.at[sh + b + NBUF]], rows[p],
                                gsem[p])

                return carry

            lax.fori_loop(0, KMAX, body, 0)
            for p in range(NBUF):
                pltpu.make_async_copy(rows[p], agg_sh.at[dst_v.at[sh]],
                                      ssem[p]).wait()
            plsc.subcore_barrier()
        copyout(nch - 1)

    return seg_sum


@functools.lru_cache(maxsize=None)
def _mlp(nch_in, dh, nch_out):
    din = nch_in * 128
    ni = N // BLK

    def body(h_ref, a_ref, w1_ref, b1_ref, g_ref, be_ref, w2_ref, b2_ref,
             w3_ref, b3_ref, o_ref, z_s, s_s, q_s):
        p = pl.program_id(0)
        i = pl.program_id(1)

        @pl.when(p == 0)
        def _():
            g = jnp.concatenate(
                [h_ref[c] + a_ref[2 * c] + a_ref[2 * c + 1]
                 for c in range(nch_in)], axis=1)
            z = jnp.dot(g, w1_ref[...], preferred_element_type=jnp.float32)
            z = z + b1_ref[...]
            z_s[pl.ds(i * BLK, BLK), :] = z
            s = jnp.sum(z, axis=0, keepdims=True)
            q = jnp.sum(z * z, axis=0, keepdims=True)

            @pl.when(i == 0)
            def _():
                s_s[...] = s
                q_s[...] = q

            @pl.when(i != 0)
            def _():
                s_s[...] += s
                q_s[...] += q

        @pl.when(p == 1)
        def _():
            mean = s_s[...] * (1.0 / N)
            var = q_s[...] * (1.0 / N) - mean * mean
            inv = g_ref[...] * lax.rsqrt(var + 1e-5)
            z = z_s[pl.ds(i * BLK, BLK), :]
            z = jnp.maximum(z * inv + (be_ref[...] - mean * inv), 0.0)
            h2 = jnp.dot(z, w2_ref[...], preferred_element_type=jnp.float32)
            h2 = jnp.maximum(h2 + b2_ref[...], 0.0)
            h3 = jnp.dot(h2, w3_ref[...], preferred_element_type=jnp.float32)
            h3 = jnp.maximum(h3 + b3_ref[...], 0.0)
            for c in range(nch_out):
                o_ref[c] = h3[:, c * 128:(c + 1) * 128]

    def first_only(p, i):
        return (0, jnp.where(p == 0, i, 0), 0)

    def const(p, i):
        return (0, 0)

    return pl.pallas_call(
        body,
        grid=(2, ni),
        in_specs=[
            pl.BlockSpec((nch_in, BLK, 128), first_only),
            pl.BlockSpec((nch_in * NC, BLK, 128), first_only),
            pl.BlockSpec((din, dh), const),
            pl.BlockSpec((1, dh), const),
            pl.BlockSpec((1, dh), const),
            pl.BlockSpec((1, dh), const),
            pl.BlockSpec((dh, dh), const),
            pl.BlockSpec((1, dh), const),
            pl.BlockSpec((dh, dh), const),
            pl.BlockSpec((1, dh), const),
        ],
        out_specs=pl.BlockSpec((nch_out, BLK, 128),
                               lambda p, i: (0, jnp.where(p == 0, 0, i), 0)),
        out_shape=jax.ShapeDtypeStruct((nch_out, N, 128), jnp.float32),
        scratch_shapes=[
            pltpu.VMEM((N, dh), jnp.float32),
            pltpu.VMEM((1, dh), jnp.float32),
            pltpu.VMEM((1, dh), jnp.float32),
        ],
    )


def _pool_head():
    def body(h_ref, b_ref, w1_ref, b1_ref, w2_ref, b2_ref, o_ref, acc):
        i = pl.program_id(0)
        bids = b_ref[0]
        oh = (lax.broadcasted_iota(jnp.int32, (NG, BLK), 0) == bids)
        oh = oh.astype(jnp.float32)
        p = jnp.dot(oh, h_ref[0], preferred_element_type=jnp.float32)

        @pl.when(i == 0)
        def _():
            acc[...] = p

        @pl.when(i != 0)
        def _():
            acc[...] += p

        @pl.when(i == N // BLK - 1)
        def _():
            t = jnp.dot(acc[...], w1_ref[...],
                        preferred_element_type=jnp.float32)
            t = jnp.maximum(t + b1_ref[...], 0.0)
            o_ref[...] = jnp.dot(t, w2_ref[...],
                                 preferred_element_type=jnp.float32) + b2_ref[...]

    return pl.pallas_call(
        body,
        grid=(N // BLK,),
        in_specs=[
            pl.BlockSpec((1, BLK, 128), lambda i: (0, i, 0)),
            pl.BlockSpec((1, 1, BLK), lambda i: (i, 0, 0)),
            pl.BlockSpec((NG, 64), lambda i: (0, 0)),
            pl.BlockSpec((1, 64), lambda i: (0, 0)),
            pl.BlockSpec((64, 128), lambda i: (0, 0)),
            pl.BlockSpec((1, 128), lambda i: (0, 0)),
        ],
        out_specs=pl.BlockSpec((NG, 128), lambda i: (0, 0)),
        out_shape=jax.ShapeDtypeStruct((NG, 128), jnp.float32),
        scratch_shapes=[pltpu.VMEM((NG, NG), jnp.float32)],
    )


_LAYER_DIMS = [(1, 128), (1, 256), (2, 512), (4, 256), (2, 128)]


def kernel(x, edge_index, batch, params):
    src = jnp.pad(edge_index[0].reshape(TB, EB), ((0, TBP - TB), (0, 0)))
    dst = jnp.pad(edge_index[1].reshape(TB, EB), ((0, TBP - TB), (0, 0)))
    zeros = jnp.zeros((N, 128), jnp.float32)
    xp = jnp.pad(x, ((0, 0), (0, 2)))
    h = xp.reshape(1, N, 128)

    for li, (nch_in, dh) in enumerate(_LAYER_DIMS):
        p = params[f"conv{li + 1}"]
        w1 = p["W1"]
        if li == 0:
            w1 = jnp.pad(w1, ((0, 2), (0, 0)))
        agg = _seg_sum(nch_in)(h, src, dst, zeros)
        h = _mlp(nch_in, dh, dh // 128)(
            h, agg.reshape(nch_in * NC, N, 128),
            w1, p["b1"].reshape(1, dh),
            p["gamma"].reshape(1, dh), p["beta"].reshape(1, dh),
            p["W2"], p["b2"].reshape(1, dh),
            p["W3"], p["b3"].reshape(1, dh))

    w2p = jnp.pad(params["lin2_W"], ((0, 0), (0, 128 - OUT)))
    b2p = jnp.pad(params["lin2_b"], ((0, 128 - OUT),)).reshape(1, 128)
    out = _pool_head()(
        h, batch.reshape(N // BLK, 1, BLK),
        params["lin1_W"], params["lin1_b"].reshape(1, 64),
        w2p, b2p)
    return out[:, :OUT]
